# baseline (speedup 1.0000x reference)
import functools

import numpy as np
import jax
import jax.numpy as jnp
from jax import lax
from jax.experimental import pallas as pl
from jax.experimental.pallas import tpu as pltpu
from jax.experimental.pallas import tpu_sc as plsc

D_MODEL = 64
MAX_LEN = 2048
SEQ_LEN = 2048
BATCH = 16

NUM_WORKERS = 32
ROWS = SEQ_LEN * BATCH
RPW = ROWS // NUM_WORKERS
CHUNK = 128
NCHUNK = RPW // CHUNK
POS_PER_CHUNK = CHUNK // BATCH
POS_PER_WORKER = RPW // BATCH
NLANE = D_MODEL // 16


def _make_pe_table(max_len, d_model):
    position = np.arange(0, max_len, dtype=np.float32)[:, None]
    div_term = np.exp(
        np.arange(0, d_model, 2).astype(np.float32) * (-np.log(10000.0) / d_model)
    )
    pe = np.zeros((max_len, d_model), dtype=np.float32)
    pe[:, 0::2] = np.sin(position * div_term)
    pe[:, 1::2] = np.cos(position * div_term)
    return pe


_PE_NP = _make_pe_table(MAX_LEN, D_MODEL)[:SEQ_LEN]

_mesh = plsc.VectorSubcoreMesh(core_axis_name="c", subcore_axis_name="s")


@functools.partial(
    pl.kernel,
    mesh=_mesh,
    compiler_params=pltpu.CompilerParams(use_tc_tiling_on_sc=False),
    out_type=jax.ShapeDtypeStruct((ROWS, D_MODEL), jnp.float32),
    scratch_types=[
        pltpu.VMEM((NCHUNK, CHUNK), jnp.int32),
        pltpu.VMEM((POS_PER_WORKER, D_MODEL), jnp.float32),
        pltpu.VMEM((CHUNK, D_MODEL), jnp.float32),
        pltpu.VMEM((CHUNK, D_MODEL), jnp.float32),
        pltpu.SemaphoreType.DMA,
        pltpu.SemaphoreType.DMA,
    ],
)
def _embed_sc(x_hbm, w_hbm, pe_hbm, out_hbm, idx_v, pe_v, buf0, buf1, gsem, osem):
    wid = lax.axis_index("s") * 2 + lax.axis_index("c")
    base_row = wid * RPW

    pltpu.sync_copy(x_hbm.at[pl.ds(wid * NCHUNK, NCHUNK)], idx_v)
    pltpu.sync_copy(pe_hbm.at[pl.ds(wid * POS_PER_WORKER, POS_PER_WORKER)], pe_v)

    bufs = (buf0, buf1)
    pltpu.async_copy(w_hbm.at[idx_v.at[0]], bufs[0], gsem).wait()
    for j in range(NCHUNK):
        buf = bufs[j % 2]
        if j + 1 < NCHUNK:
            nxt = pltpu.async_copy(w_hbm.at[idx_v.at[j + 1]], bufs[(j + 1) % 2], gsem)

        def body(p, _):
            prow = j * POS_PER_CHUNK + p
            pes = [pe_v[prow, pl.ds(c * 16, 16)] for c in range(NLANE)]
            for b in range(BATCH):
                r = p * BATCH + b
                for c in range(NLANE):
                    sl = pl.ds(c * 16, 16)
                    buf[r, sl] = buf[r, sl] + pes[c]
            return 0

        lax.fori_loop(0, POS_PER_CHUNK, body, 0)
        pltpu.async_copy(
            buf, out_hbm.at[pl.ds(base_row + j * CHUNK, CHUNK)], osem
        ).wait()
        if j + 1 < NCHUNK:
            nxt.wait()


def kernel(x, W):
    xr = x.reshape(ROWS // CHUNK, CHUNK)
    out = _embed_sc(xr, W, jnp.asarray(_PE_NP))
    return out.reshape(SEQ_LEN, BATCH, D_MODEL)


# speedup vs baseline: 1.0255x; 1.0255x over previous
import functools

import numpy as np
import jax
import jax.numpy as jnp
from jax import lax
from jax.experimental import pallas as pl
from jax.experimental.pallas import tpu as pltpu
from jax.experimental.pallas import tpu_sc as plsc

D_MODEL = 64
MAX_LEN = 2048
SEQ_LEN = 2048
BATCH = 16

NUM_WORKERS = 32
ROWS = SEQ_LEN * BATCH
RPW = ROWS // NUM_WORKERS
CHUNK = 128
NCHUNK = RPW // CHUNK
POS_PER_CHUNK = CHUNK // BATCH
POS_PER_WORKER = RPW // BATCH
NLANE = D_MODEL // 16
NBUF = 4


def _make_pe_table(max_len, d_model):
    position = np.arange(0, max_len, dtype=np.float32)[:, None]
    div_term = np.exp(
        np.arange(0, d_model, 2).astype(np.float32) * (-np.log(10000.0) / d_model)
    )
    pe = np.zeros((max_len, d_model), dtype=np.float32)
    pe[:, 0::2] = np.sin(position * div_term)
    pe[:, 1::2] = np.cos(position * div_term)
    return pe


_PE_NP = _make_pe_table(MAX_LEN, D_MODEL)[:SEQ_LEN]

_mesh = plsc.VectorSubcoreMesh(core_axis_name="c", subcore_axis_name="s")


@functools.partial(
    pl.kernel,
    mesh=_mesh,
    compiler_params=pltpu.CompilerParams(use_tc_tiling_on_sc=False),
    out_type=jax.ShapeDtypeStruct((ROWS, D_MODEL), jnp.float32),
    scratch_types=[
        pltpu.VMEM((POS_PER_WORKER, BATCH), jnp.int32),
        pltpu.VMEM((NCHUNK, CHUNK), jnp.int32),
        pltpu.VMEM((POS_PER_WORKER, D_MODEL), jnp.float32),
        pltpu.VMEM((NBUF, CHUNK, D_MODEL), jnp.float32),
        pltpu.SemaphoreType.DMA((NBUF,)),
        pltpu.SemaphoreType.DMA((NBUF,)),
    ],
)
def _embed_sc(x_hbm, w_hbm, pe_hbm, out_hbm, idxa, idxb, pe_v, bufs, gsem, osem):
    wid = lax.axis_index("s") * 2 + lax.axis_index("c")
    base_row = wid * RPW
    base_pos = wid * POS_PER_WORKER

    pltpu.sync_copy(x_hbm.at[pl.ds(base_pos, POS_PER_WORKER)], idxa)
    for j in range(NCHUNK):
        for t in range(POS_PER_CHUNK):
            idxb[j, pl.ds(t * BATCH, BATCH)] = idxa[j * POS_PER_CHUNK + t, :]
    pltpu.sync_copy(pe_hbm.at[pl.ds(base_pos, POS_PER_WORKER)], pe_v)

    def fire_gather(j):
        return pltpu.async_copy(
            w_hbm.at[idxb.at[j]], bufs.at[j % NBUF], gsem.at[j % NBUF]
        )

    gathers = [fire_gather(0), fire_gather(1)]
    writebacks = [None] * NCHUNK
    for j in range(NCHUNK):
        b = j % NBUF
        if j + 2 < NCHUNK:
            if j - 2 >= 0:
                writebacks[j - 2].wait()
            gathers.append(fire_gather(j + 2))
        gathers[j].wait()

        def body(p, _):
            prow = j * POS_PER_CHUNK + p
            pes = [pe_v[prow, pl.ds(c * 16, 16)] for c in range(NLANE)]
            for t in range(BATCH):
                r = p * BATCH + t
                for c in range(NLANE):
                    sl = pl.ds(c * 16, 16)
                    bufs[b, r, sl] = bufs[b, r, sl] + pes[c]
            return 0

        lax.fori_loop(0, POS_PER_CHUNK, body, 0)
        writebacks[j] = pltpu.async_copy(
            bufs.at[b], out_hbm.at[pl.ds(base_row + j * CHUNK, CHUNK)], osem.at[b]
        )
    for j in range(NCHUNK - 4, NCHUNK):
        writebacks[j].wait()


def kernel(x, W):
    out = _embed_sc(x, W, jnp.asarray(_PE_NP))
    return out.reshape(SEQ_LEN, BATCH, D_MODEL)
